# baseline (speedup 1.0000x reference)
import jax
import jax.numpy as jnp
from jax.experimental import pallas as pl

_B = 34


def _argmax_body(x_ref, o_ref):
    nb, h, w = x_ref.shape
    ch = 32
    r = ch // 8
    nc = h // ch
    big = jnp.int32(1 << 20)

    acc = jnp.max(x_ref[...].reshape(nb, h // 8, 8, w), axis=1)
    m = jnp.max(acc, axis=(1, 2))
    mb = m[:, None, None, None]

    jj = jax.lax.broadcasted_iota(jnp.int32, (1, r, 8, w), 1)
    ss = jax.lax.broadcasted_iota(jnp.int32, (1, r, 8, w), 2)
    rowrel = jj * 8 + ss
    best8 = None
    for i in range(nc):
        c4 = x_ref[:, i * ch:(i + 1) * ch, :].reshape(nb, r, 8, w)
        rel = jnp.min(jnp.where(c4 == mb, rowrel, big), axis=1) + i * ch
        best8 = rel if best8 is None else jnp.minimum(best8, rel)

    col = jax.lax.broadcasted_iota(jnp.int32, (1, 8, w), 2)
    idx = jnp.min(jnp.where(best8 < h, best8 * w + col, big), axis=(1, 2))
    wi = idx % w
    hi = idx // w
    sel = jax.lax.broadcasted_iota(jnp.int32, (1, 1, 2), 2)
    o_ref[...] = jnp.where(sel == 0, wi[:, None, None], hi[:, None, None])


def kernel(heatmaps):
    b, p, h, w = heatmaps.shape
    n = b * p
    flat = heatmaps.reshape(n, h, w)
    out = pl.pallas_call(
        _argmax_body,
        grid=(n // _B,),
        in_specs=[pl.BlockSpec((_B, h, w), lambda i: (i, 0, 0))],
        out_specs=pl.BlockSpec((_B, 1, 2), lambda i: (i, 0, 0)),
        out_shape=jax.ShapeDtypeStruct((n, 1, 2), jnp.int32),
    )(flat)
    return out.reshape(b, p, 2)


# speedup vs baseline: 1.0028x; 1.0028x over previous
import jax
import jax.numpy as jnp
from jax.experimental import pallas as pl

_B = 32


def _argmax_body(x_ref, o_ref):
    nb, h, w = x_ref.shape
    ch = 32
    r = ch // 8
    nc = h // ch
    big = jnp.int32(1 << 20)

    acc = jnp.max(x_ref[...].reshape(nb, h // 8, 8, w), axis=1)
    m = jnp.max(acc, axis=(1, 2))
    mb = m[:, None, None, None]

    jj = jax.lax.broadcasted_iota(jnp.int32, (1, r, 8, w), 1)
    ss = jax.lax.broadcasted_iota(jnp.int32, (1, r, 8, w), 2)
    rowrel = jj * 8 + ss
    best8 = None
    for i in range(nc):
        c4 = x_ref[:, i * ch:(i + 1) * ch, :].reshape(nb, r, 8, w)
        rel = jnp.min(jnp.where(c4 == mb, rowrel, big), axis=1) + i * ch
        best8 = rel if best8 is None else jnp.minimum(best8, rel)

    col = jax.lax.broadcasted_iota(jnp.int32, (1, 8, w), 2)
    idx = jnp.min(jnp.where(best8 < h, best8 * w + col, big), axis=(1, 2))
    wi = idx % w
    hi = idx // w
    sel = jax.lax.broadcasted_iota(jnp.int32, (1, 1, 2), 2)
    o_ref[...] = jnp.where(sel == 0, wi[:, None, None], hi[:, None, None])


def kernel(heatmaps):
    b, p, h, w = heatmaps.shape
    n = b * p
    flat = heatmaps.reshape(n, h, w)
    out = pl.pallas_call(
        _argmax_body,
        grid=(n // _B,),
        in_specs=[pl.BlockSpec((_B, h, w), lambda i: (i, 0, 0))],
        out_specs=pl.BlockSpec((_B, 1, 2), lambda i: (i, 0, 0)),
        out_shape=jax.ShapeDtypeStruct((n, 1, 2), jnp.int32),
    )(flat)
    return out.reshape(b, p, 2)
